# baseline (speedup 1.0000x reference)
import functools

import jax
import jax.numpy as jnp
from jax import lax
from jax.experimental import pallas as pl
from jax.experimental.pallas import tpu as pltpu
from jax.experimental.pallas import tpu_sc as plsc

F32 = jnp.float32
I32 = jnp.int32

N = 10000
E = 320000
G = 50
N0 = 200
K1, K2, K3 = 160, 128, 103

NWORK = 32
EPAD = 327680
CH = 128
R1 = 10240
R2 = 8192
R3 = 8192
D1, D2, D3 = N, G * K1, G * K2


def _sc_segsum(table, ns, nd, counts):
    R, W = table.shape
    epw = ns.shape[0] // NWORK
    rpt = R // 16
    rch = rpt // CH
    mesh = plsc.VectorSubcoreMesh(core_axis_name="c", subcore_axis_name="s",
                                  num_cores=2, num_subcores=16)

    @functools.partial(
        pl.kernel,
        out_type=jax.ShapeDtypeStruct((2, R, W), F32),
        mesh=mesh,
        compiler_params=pltpu.CompilerParams(needs_layout_passes=False,
                                             use_tc_tiling_on_sc=False),
        scratch_types=[
            pltpu.VMEM((CH,), I32),
            pltpu.VMEM((CH,), I32),
            pltpu.VMEM((CH,), I32),
            pltpu.VMEM((CH,), I32),
            pltpu.VMEM((CH, W), F32),
            pltpu.VMEM((CH, W), F32),
            pltpu.VMEM((16,), I32),
            pltpu.VMEM_SHARED((R, W), F32),
            pltpu.SemaphoreType.DMA,
            pltpu.SemaphoreType.DMA,
        ],
    )
    def k(table_h, ns_h, nd_h, counts_h, out_h, ids0, idd0, ids1, idd1,
          rows0, rows1, cbuf, acc, sem0, sem1):
        cid = lax.axis_index("c")
        sid = lax.axis_index("s")
        wid = cid * 16 + sid

        def zrow(r, _):
            for c in range(W // 16):
                rows0[r, pl.ds(c * 16, 16)] = jnp.zeros((16,), F32)
            return 0
        lax.fori_loop(0, CH, zrow, 0)

        def zacc(i, _):
            pltpu.sync_copy(rows0, acc.at[pl.ds(sid * rpt + i * CH, CH), :])
            return 0
        lax.fori_loop(0, rch, zacc, 0)
        plsc.subcore_barrier()

        pltpu.sync_copy(counts_h.at[wid], cbuf)
        cntp = jnp.max(cbuf[...])
        base0 = wid * epw
        last = pl.multiple_of(base0 + cntp - CH, CH)

        pltpu.sync_copy(ns_h.at[pl.ds(base0, CH)], ids0)
        pltpu.sync_copy(nd_h.at[pl.ds(base0, CH)], idd0)
        gat0 = pltpu.async_copy(table_h.at[ids0], rows0, sem0)

        def body(i, _):
            b1 = pl.multiple_of(
                jnp.minimum(base0 + (2 * i + 1) * CH, last), CH)
            pltpu.sync_copy(ns_h.at[pl.ds(b1, CH)], ids1)
            pltpu.sync_copy(nd_h.at[pl.ds(b1, CH)], idd1)
            g1 = pltpu.async_copy(table_h.at[ids1], rows1, sem1)
            pltpu.make_async_copy(table_h.at[ids0], rows0, sem0).wait()
            pltpu.sync_copy(rows0, acc.at[idd0], add=True)
            b2 = pl.multiple_of(
                jnp.minimum(base0 + (2 * i + 2) * CH, last), CH)
            pltpu.sync_copy(ns_h.at[pl.ds(b2, CH)], ids0)
            pltpu.sync_copy(nd_h.at[pl.ds(b2, CH)], idd0)
            pltpu.async_copy(table_h.at[ids0], rows0, sem0)
            g1.wait()
            pltpu.sync_copy(rows1, acc.at[idd1], add=True)
            return 0
        lax.fori_loop(0, cntp // (2 * CH), body, 0)
        pltpu.make_async_copy(table_h.at[ids0], rows0, sem0).wait()
        plsc.subcore_barrier()

        def cout(i, _):
            off = sid * rpt + i * CH
            pltpu.sync_copy(acc.at[pl.ds(off, CH), :],
                            out_h.at[cid, pl.ds(off, CH), :])
            return 0
        lax.fori_loop(0, rch, cout, 0)

    return k(table, ns, nd, counts)


def _sc_pool_apply(hsel, inv, mapping, esrc, edst, counts_in, rn,
                   dummy_next):
    _, W = hsel.shape
    M = mapping.shape[0]
    epad = esrc.shape[0]
    epw = epad // NWORK
    EB = 1024
    nch = epw // EB
    spw = rn // NWORK
    mesh = plsc.VectorSubcoreMesh(core_axis_name="c", subcore_axis_name="s",
                                  num_cores=2, num_subcores=16)

    @functools.partial(
        pl.kernel,
        out_type=[
            jax.ShapeDtypeStruct((rn, W), F32),
            jax.ShapeDtypeStruct((epad,), I32),
            jax.ShapeDtypeStruct((epad,), I32),
            jax.ShapeDtypeStruct((NWORK, 16), I32),
        ],
        mesh=mesh,
        compiler_params=pltpu.CompilerParams(needs_layout_passes=False,
                                             use_tc_tiling_on_sc=False),
        scratch_types=[
            pltpu.VMEM((CH,), I32),
            pltpu.VMEM((CH,), I32),
            pltpu.VMEM((CH, W), F32),
            pltpu.VMEM((CH, W), F32),
            pltpu.VMEM((M,), I32),
            pltpu.VMEM((EB,), I32),
            pltpu.VMEM((EB,), I32),
            pltpu.VMEM((epw + 2 * CH,), I32),
            pltpu.VMEM((epw + 2 * CH,), I32),
            pltpu.VMEM((16,), I32),
            pltpu.SemaphoreType.DMA,
            pltpu.SemaphoreType.DMA,
        ],
    )
    def k(hsel_h, inv_h, map_h, es_h, ed_h, cin_h, xt_h, ns_h, nd_h, cout_h,
          idxb0, idxb1, rowb0, rowb1, mapv, eins, eind, outs, outd, cbuf,
          sem0, sem1):
        cid = lax.axis_index("c")
        sid = lax.axis_index("s")
        wid = cid * 16 + sid

        sbase = wid * spw
        pltpu.sync_copy(inv_h.at[pl.ds(sbase, CH)], idxb0)
        pltpu.sync_copy(inv_h.at[pl.ds(sbase + CH, CH)], idxb1)
        g0 = pltpu.async_copy(hsel_h.at[idxb0], rowb0, sem0)
        g1 = pltpu.async_copy(hsel_h.at[idxb1], rowb1, sem1)
        pltpu.sync_copy(map_h, mapv)
        pltpu.sync_copy(cin_h.at[wid], cbuf)
        cin = jnp.max(cbuf[...])
        g0.wait()
        pltpu.sync_copy(rowb0, xt_h.at[pl.ds(sbase, CH), :])
        g1.wait()
        pltpu.sync_copy(rowb1, xt_h.at[pl.ds(sbase + CH, CH), :])

        ebase = wid * epw
        iot = lax.iota(I32, 16)

        def ebody(g, cnt):
            b = ebase + g * EB
            pltpu.sync_copy(es_h.at[pl.ds(b, EB)], eins)
            pltpu.sync_copy(ed_h.at[pl.ds(b, EB)], eind)
            for c in range(EB // 16):
                ivs = jnp.clip(eins[pl.ds(c * 16, 16)], 0, M - 1)
                ivd = jnp.clip(eind[pl.ds(c * 16, 16)], 0, M - 1)
                ms = plsc.load_gather(mapv, [ivs])
                md = plsc.load_gather(mapv, [ivd])
                pos = g * EB + c * 16 + iot
                valid = (ms >= 0) & (md >= 0) & (pos < cin)
                tgt = cnt - 1 + plsc.cumsum(valid.astype(I32))
                plsc.store_scatter(outs, [tgt], ms, mask=valid)
                plsc.store_scatter(outd, [tgt], md, mask=valid)
                cnt = cnt + jnp.max(plsc.all_reduce_population_count(valid))
            return cnt
        cnt = lax.fori_loop(0, nch, ebody, jnp.int32(0))

        cpad = jnp.maximum(((cnt + 2 * CH - 1) // (2 * CH)) * (2 * CH),
                           2 * CH)

        def fbody(i, _):
            off = cnt + i * 16
            dvec = dummy_next + (off + iot) % CH
            plsc.store_scatter(outs, [off + iot], dvec)
            plsc.store_scatter(outd, [off + iot], dvec)
            return 0
        lax.fori_loop(0, (cpad - cnt + 15) // 16, fbody, 0)

        pltpu.sync_copy(outs.at[pl.ds(0, epw)], ns_h.at[pl.ds(ebase, epw)])
        pltpu.sync_copy(outd.at[pl.ds(0, epw)], nd_h.at[pl.ds(ebase, epw)])
        cbuf[...] = jnp.full((16,), 1, I32) * cpad
        pltpu.sync_copy(cbuf, cout_h.at[wid])

    return k(hsel, inv, mapping, esrc, edst, counts_in)


def _tc_layer(p0, p1, x, wr_t, wx_t, b, wp, n, k, kpad=256):
    _, wx = x.shape
    wa = wr_t.shape[0]

    def body(p0_r, p1_r, x_r, wr_r, wx_r, b_r, wp_r,
             map_r, inv_r, ro_r, hs_r):
        g = pl.program_id(0)
        agg = p0_r[...] + p1_r[...]
        hv = ((jnp.dot(agg, wr_r[...], preferred_element_type=F32)
               + b_r[...])
              + jnp.dot(x_r[...], wx_r[...], preferred_element_type=F32))
        hv = jnp.maximum(hv, 0.0)
        wpv = wp_r[...]
        nrm = jnp.sqrt(jnp.sum(wpv * wpv))
        scol = jnp.tanh(jnp.dot(hv, wpv, preferred_element_type=F32)
                        / nrm)
        srow = jnp.reshape(scol, (1, n))

        ii = lax.broadcasted_iota(I32, (n, n), 0)
        jj = lax.broadcasted_iota(I32, (n, n), 1)
        gt = (srow > scol)
        eq = (srow == scol)
        rank_c = jnp.sum(gt.astype(F32) + (eq & (jj < ii)).astype(F32),
                         axis=1, keepdims=True)
        sel_c = rank_c < k
        mapv = jnp.where(sel_c, g * k + rank_c.astype(I32), -1)
        map_r[...] = mapv

        tt = lax.broadcasted_iota(I32, (n, kpad), 1)
        onehot = (mapv == g * k + tt)
        jglob = (lax.broadcasted_iota(I32, (n, kpad), 0)
                 + g * n).astype(F32)
        inv_r[...] = jnp.sum(jnp.where(onehot, jglob, 0.0),
                             axis=0, keepdims=True).astype(I32)[None]

        scaled = hv * scol
        hs = jnp.where(sel_c, scaled, 0.0)
        hs_r[...] = hs
        maxv = jnp.max(jnp.where(sel_c, scaled, -1e30), axis=0,
                       keepdims=True)
        meanv = jnp.sum(hs, axis=0, keepdims=True) * (1.0 / k)
        ro_r[...] = jnp.concatenate([maxv, meanv], axis=1)[None]

    mapc, inv3, ro3, hs = pl.pallas_call(
        body,
        grid=(G,),
        in_specs=[
            pl.BlockSpec((n, wa), lambda g: (g, 0)),
            pl.BlockSpec((n, wa), lambda g: (g, 0)),
            pl.BlockSpec((n, wx), lambda g: (g, 0)),
            pl.BlockSpec((wa, 128), lambda g: (0, 0)),
            pl.BlockSpec((wx, 128), lambda g: (0, 0)),
            pl.BlockSpec((1, 128), lambda g: (0, 0)),
            pl.BlockSpec((128, 1), lambda g: (0, 0)),
        ],
        out_specs=[
            pl.BlockSpec((n, 1), lambda g: (g, 0)),
            pl.BlockSpec((1, 1, kpad), lambda g: (g, 0, 0)),
            pl.BlockSpec((1, 1, 256), lambda g: (g, 0, 0)),
            pl.BlockSpec((n, 128), lambda g: (g, 0)),
        ],
        out_shape=[
            jax.ShapeDtypeStruct((G * n, 1), I32),
            jax.ShapeDtypeStruct((G, 1, kpad), I32),
            jax.ShapeDtypeStruct((G, 1, 256), F32),
            jax.ShapeDtypeStruct((G * n, 128), F32),
        ],
    )(p0, p1, x, wr_t, wx_t, b, wp)
    return mapc, inv3[:, 0], ro3[:, 0], hs


def _tc_head(r1, r2, r3, w1t, b1, w2t, b2, w3t, b3):
    def body(r1_r, r2_r, r3_r, w1_r, b1_r, w2_r, b2_r, w3_r, b3_r, o_r):
        z = r1_r[...] + r2_r[...] + r3_r[...]
        z = jnp.maximum(jnp.dot(z, w1_r[...], preferred_element_type=F32)
                        + b1_r[...], 0.0)
        z = jnp.maximum(jnp.dot(z, w2_r[...], preferred_element_type=F32)
                        + b2_r[...], 0.0)
        z3 = jnp.dot(z, w3_r[...], preferred_element_type=F32) + b3_r[...]
        col = lax.broadcasted_iota(I32, (64, 128), 1)
        logits = jnp.where(col < 7, z3, -1e30)
        m = jnp.max(logits, axis=1, keepdims=True)
        lse = jnp.log(jnp.sum(jnp.exp(logits - m), axis=1,
                              keepdims=True)) + m
        o_r[...] = logits - lse

    def full(shape):
        return pl.BlockSpec(shape, lambda: (0, 0))

    return pl.pallas_call(
        body,
        in_specs=[full((64, 256)), full((64, 256)), full((64, 256)),
                  full((256, 128)), full((1, 128)),
                  full((128, 128)), full((1, 128)),
                  full((128, 128)), full((1, 128))],
        out_specs=full((64, 128)),
        out_shape=jax.ShapeDtypeStruct((64, 128), F32),
    )(r1, r2, r3, w1t, b1, w2t, b2, w3t, b3)


def kernel(x, edge_index, batch, W_rel1, b_rel1, W_root1, w_pool1,
           W_rel2, b_rel2, W_root2, w_pool2, W_rel3, b_rel3, W_root3,
           lin1_w, lin1_b, lin2_w, lin2_b, lin3_w, lin3_b):
    pad_ids = D1 + (jnp.arange(EPAD - E, dtype=I32) % 128)
    src = jnp.concatenate([edge_index[0], pad_ids])
    dst = jnp.concatenate([edge_index[1], pad_ids])

    x1t = jnp.zeros((R1, 16), F32).at[:N, :4].set(x)
    wr1 = jnp.zeros((16, 128), F32).at[:4].set(W_rel1.T)
    wx1 = jnp.zeros((16, 128), F32).at[:4].set(W_root1.T)
    cfull = jnp.full((NWORK, 16), EPAD // NWORK, I32)
    parts1 = _sc_segsum(x1t, src, dst, cfull)
    map1, inv1, ro1, hs1 = _tc_layer(
        parts1[0, :N], parts1[1, :N], x1t[:N], wr1, wx1,
        b_rel1[None, :], w_pool1[:, None], n=N0, k=K1)
    mapping1 = jnp.full((R1,), -1, I32).at[:N].set(map1[:, 0])
    invf1 = jnp.concatenate([inv1[:, :K1].reshape(-1),
                             jnp.full((R2 - G * K1,), N, I32)])
    hs1p = jnp.zeros((R1, 128), F32).at[:N].set(hs1)

    x2t, ns2, nd2, cnt2 = _sc_pool_apply(hs1p, invf1, mapping1, src, dst,
                                         cfull, rn=R2, dummy_next=D2)
    parts2 = _sc_segsum(x2t, ns2, nd2, cnt2)
    map2, inv2, ro2, hs2 = _tc_layer(
        parts2[0, :G * K1], parts2[1, :G * K1], x2t[:G * K1],
        W_rel2.T, W_root2.T, b_rel2[None, :], w_pool2[:, None], n=K1, k=K2)
    mapping2 = jnp.full((R2,), -1, I32).at[:G * K1].set(map2[:, 0])
    invf2 = jnp.concatenate([inv2[:, :K2].reshape(-1),
                             jnp.full((R3 - G * K2,), D2, I32)])
    hs2p = jnp.zeros((R2, 128), F32).at[:G * K1].set(hs2)

    x3t, ns3, nd3, cnt3 = _sc_pool_apply(hs2p, invf2, mapping2, ns2, nd2,
                                         cnt2, rn=R3, dummy_next=D3)
    parts3 = _sc_segsum(x3t, ns3, nd3, cnt3)
    _, _, ro3, _ = _tc_layer(
        parts3[0, :G * K2], parts3[1, :G * K2], x3t[:G * K2],
        W_rel3.T, W_root3.T, b_rel3[None, :], w_pool2[:, None], n=K2, k=K3)

    def pad64(r):
        return jnp.zeros((64, 256), F32).at[:G].set(r)

    w2t = jnp.zeros((128, 128), F32).at[:, :64].set(lin2_w.T)
    b2p = jnp.zeros((1, 128), F32).at[0, :64].set(lin2_b)
    w3t = jnp.zeros((128, 128), F32).at[:64, :7].set(lin3_w.T)
    b3p = jnp.zeros((1, 128), F32).at[0, :7].set(lin3_b)
    out = _tc_head(pad64(ro1), pad64(ro2), pad64(ro3),
                   lin1_w.T, lin1_b[None, :], w2t, b2p, w3t, b3p)
    return out[:G, :7]


# speedup vs baseline: 2.4433x; 2.4433x over previous
import functools

import jax
import jax.numpy as jnp
from jax import lax
from jax.experimental import pallas as pl
from jax.experimental.pallas import tpu as pltpu
from jax.experimental.pallas import tpu_sc as plsc

F32 = jnp.float32
I32 = jnp.int32

N = 10000
E = 320000
G = 50
N0 = 200
K1, K2, K3 = 160, 128, 103

NWORK = 32
EPAD = 327680
CH = 128
R1 = 10240
R2 = 8192
R3 = 8192
D1, D2, D3 = N, G * K1, G * K2


def _sc_segsum(table, ns, nd, counts):
    R, W = table.shape
    epw = ns.shape[0] // NWORK
    rpt = R // 16
    rch = rpt // CH
    mesh = plsc.VectorSubcoreMesh(core_axis_name="c", subcore_axis_name="s",
                                  num_cores=2, num_subcores=16)

    @functools.partial(
        pl.kernel,
        out_type=jax.ShapeDtypeStruct((2, R, W), F32),
        mesh=mesh,
        compiler_params=pltpu.CompilerParams(needs_layout_passes=False,
                                             use_tc_tiling_on_sc=False),
        scratch_types=[
            pltpu.VMEM((CH,), I32),
            pltpu.VMEM((CH,), I32),
            pltpu.VMEM((CH,), I32),
            pltpu.VMEM((CH,), I32),
            pltpu.VMEM((CH, W), F32),
            pltpu.VMEM((CH, W), F32),
            pltpu.VMEM((16,), I32),
            pltpu.VMEM_SHARED((R, W), F32),
            pltpu.SemaphoreType.DMA,
            pltpu.SemaphoreType.DMA,
        ],
    )
    def k(table_h, ns_h, nd_h, counts_h, out_h, ids0, idd0, ids1, idd1,
          rows0, rows1, cbuf, acc, sem0, sem1):
        cid = lax.axis_index("c")
        sid = lax.axis_index("s")
        wid = cid * 16 + sid

        def zrow(r, _):
            for c in range(W // 16):
                rows0[r, pl.ds(c * 16, 16)] = jnp.zeros((16,), F32)
            return 0
        lax.fori_loop(0, CH, zrow, 0)

        def zacc(i, _):
            pltpu.sync_copy(rows0, acc.at[pl.ds(sid * rpt + i * CH, CH), :])
            return 0
        lax.fori_loop(0, rch, zacc, 0)
        plsc.subcore_barrier()

        pltpu.sync_copy(counts_h.at[wid], cbuf)
        cntp = jnp.max(cbuf[...])
        base0 = wid * epw
        last = pl.multiple_of(base0 + cntp - CH, CH)

        pltpu.sync_copy(ns_h.at[pl.ds(base0, CH)], ids0)
        pltpu.sync_copy(nd_h.at[pl.ds(base0, CH)], idd0)
        gat0 = pltpu.async_copy(table_h.at[ids0], rows0, sem0)

        def body(i, _):
            b1 = pl.multiple_of(
                jnp.minimum(base0 + (2 * i + 1) * CH, last), CH)
            pltpu.sync_copy(ns_h.at[pl.ds(b1, CH)], ids1)
            pltpu.sync_copy(nd_h.at[pl.ds(b1, CH)], idd1)
            g1 = pltpu.async_copy(table_h.at[ids1], rows1, sem1)
            pltpu.make_async_copy(table_h.at[ids0], rows0, sem0).wait()
            pltpu.sync_copy(rows0, acc.at[idd0], add=True)
            b2 = pl.multiple_of(
                jnp.minimum(base0 + (2 * i + 2) * CH, last), CH)
            pltpu.sync_copy(ns_h.at[pl.ds(b2, CH)], ids0)
            pltpu.sync_copy(nd_h.at[pl.ds(b2, CH)], idd0)
            pltpu.async_copy(table_h.at[ids0], rows0, sem0)
            g1.wait()
            pltpu.sync_copy(rows1, acc.at[idd1], add=True)
            return 0
        lax.fori_loop(0, cntp // (2 * CH), body, 0)
        pltpu.make_async_copy(table_h.at[ids0], rows0, sem0).wait()
        plsc.subcore_barrier()

        def cout(i, _):
            off = sid * rpt + i * CH
            pltpu.sync_copy(acc.at[pl.ds(off, CH), :],
                            out_h.at[cid, pl.ds(off, CH), :])
            return 0
        lax.fori_loop(0, rch, cout, 0)

    return k(table, ns, nd, counts)


def _sc_pool_apply(hsel, inv, mapping, esrc, edst, counts_in, rn,
                   dummy_next):
    _, W = hsel.shape
    M = mapping.shape[0]
    epad = esrc.shape[0]
    epw = epad // NWORK
    EB = 1024
    nch = epw // EB
    spw = rn // NWORK
    mesh = plsc.VectorSubcoreMesh(core_axis_name="c", subcore_axis_name="s",
                                  num_cores=2, num_subcores=16)

    @functools.partial(
        pl.kernel,
        out_type=[
            jax.ShapeDtypeStruct((rn, W), F32),
            jax.ShapeDtypeStruct((epad,), I32),
            jax.ShapeDtypeStruct((epad,), I32),
            jax.ShapeDtypeStruct((NWORK, 16), I32),
        ],
        mesh=mesh,
        compiler_params=pltpu.CompilerParams(needs_layout_passes=False,
                                             use_tc_tiling_on_sc=False),
        scratch_types=[
            pltpu.VMEM((CH,), I32),
            pltpu.VMEM((CH,), I32),
            pltpu.VMEM((CH, W), F32),
            pltpu.VMEM((CH, W), F32),
            pltpu.VMEM((M,), I32),
            pltpu.VMEM((EB,), I32),
            pltpu.VMEM((EB,), I32),
            pltpu.VMEM((epw + 2 * CH,), I32),
            pltpu.VMEM((epw + 2 * CH,), I32),
            pltpu.VMEM((16,), I32),
            pltpu.SemaphoreType.DMA,
            pltpu.SemaphoreType.DMA,
        ],
    )
    def k(hsel_h, inv_h, map_h, es_h, ed_h, cin_h, xt_h, ns_h, nd_h, cout_h,
          idxb0, idxb1, rowb0, rowb1, mapv, eins, eind, outs, outd, cbuf,
          sem0, sem1):
        cid = lax.axis_index("c")
        sid = lax.axis_index("s")
        wid = cid * 16 + sid

        sbase = wid * spw
        pltpu.sync_copy(inv_h.at[pl.ds(sbase, CH)], idxb0)
        pltpu.sync_copy(inv_h.at[pl.ds(sbase + CH, CH)], idxb1)
        g0 = pltpu.async_copy(hsel_h.at[idxb0], rowb0, sem0)
        g1 = pltpu.async_copy(hsel_h.at[idxb1], rowb1, sem1)
        pltpu.sync_copy(map_h, mapv)
        pltpu.sync_copy(cin_h.at[wid], cbuf)
        cin = jnp.max(cbuf[...])
        g0.wait()
        pltpu.sync_copy(rowb0, xt_h.at[pl.ds(sbase, CH), :])
        g1.wait()
        pltpu.sync_copy(rowb1, xt_h.at[pl.ds(sbase + CH, CH), :])

        ebase = wid * epw
        iot = lax.iota(I32, 16)

        def ebody(g, cnt):
            b = ebase + g * EB
            pltpu.sync_copy(es_h.at[pl.ds(b, EB)], eins)
            pltpu.sync_copy(ed_h.at[pl.ds(b, EB)], eind)
            for c in range(EB // 16):
                ivs = jnp.clip(eins[pl.ds(c * 16, 16)], 0, M - 1)
                ivd = jnp.clip(eind[pl.ds(c * 16, 16)], 0, M - 1)
                ms = plsc.load_gather(mapv, [ivs])
                md = plsc.load_gather(mapv, [ivd])
                pos = g * EB + c * 16 + iot
                valid = (ms >= 0) & (md >= 0) & (pos < cin)
                tgt = cnt - 1 + plsc.cumsum(valid.astype(I32))
                plsc.store_scatter(outs, [tgt], ms, mask=valid)
                plsc.store_scatter(outd, [tgt], md, mask=valid)
                cnt = cnt + jnp.max(plsc.all_reduce_population_count(valid))
            return cnt
        cnt = lax.fori_loop(0, (cin + EB - 1) // EB, ebody,
                            jnp.int32(0))

        cpad = jnp.maximum(((cnt + 2 * CH - 1) // (2 * CH)) * (2 * CH),
                           2 * CH)

        def fbody(i, _):
            off = cnt + i * 16
            dvec = dummy_next + (off + iot) % CH
            plsc.store_scatter(outs, [off + iot], dvec)
            plsc.store_scatter(outd, [off + iot], dvec)
            return 0
        lax.fori_loop(0, (cpad - cnt + 15) // 16, fbody, 0)

        pltpu.sync_copy(outs.at[pl.ds(0, epw)], ns_h.at[pl.ds(ebase, epw)])
        pltpu.sync_copy(outd.at[pl.ds(0, epw)], nd_h.at[pl.ds(ebase, epw)])
        cbuf[...] = jnp.full((16,), 1, I32) * cpad
        pltpu.sync_copy(cbuf, cout_h.at[wid])

    return k(hsel, inv, mapping, esrc, edst, counts_in)


def _tc_conv(p0, p1, x, wr_t, wx_t, b, wp):
    np_, wx = x.shape
    wa = wr_t.shape[0]
    blk = 256
    grid = np_ // blk

    def body(p0_r, p1_r, x_r, wr_r, wx_r, b_r, wp_r, h_r, s_r):
        agg = p0_r[...] + p1_r[...]
        h = ((jnp.dot(agg, wr_r[...], preferred_element_type=F32)
              + b_r[...])
             + jnp.dot(x_r[...], wx_r[...], preferred_element_type=F32))
        h = jnp.maximum(h, 0.0)
        h_r[...] = h
        wpv = wp_r[...]
        nrm = jnp.sqrt(jnp.sum(wpv * wpv))
        s_r[...] = jnp.tanh(jnp.dot(h, wpv, preferred_element_type=F32)
                            / nrm)

    return pl.pallas_call(
        body,
        grid=(grid,),
        in_specs=[
            pl.BlockSpec((blk, wa), lambda i: (i, 0)),
            pl.BlockSpec((blk, wa), lambda i: (i, 0)),
            pl.BlockSpec((blk, wx), lambda i: (i, 0)),
            pl.BlockSpec((wa, 128), lambda i: (0, 0)),
            pl.BlockSpec((wx, 128), lambda i: (0, 0)),
            pl.BlockSpec((1, 128), lambda i: (0, 0)),
            pl.BlockSpec((128, 1), lambda i: (0, 0)),
        ],
        out_specs=[
            pl.BlockSpec((blk, 128), lambda i: (i, 0)),
            pl.BlockSpec((blk, 1), lambda i: (i, 0)),
        ],
        out_shape=[
            jax.ShapeDtypeStruct((np_, 128), F32),
            jax.ShapeDtypeStruct((np_, 1), F32),
        ],
    )(p0, p1, x, wr_t, wx_t, b, wp)


def _tc_pool(s_row, s_col, h, n, npad, k, kpad=256):
    def body(sp_r, sc_r, h_r, map_r, inv_r, ro_r, hs_r):
        g = pl.program_id(0)
        srow = sp_r[0]
        scol = sc_r[...]
        hv = h_r[...]

        ii = lax.broadcasted_iota(I32, (n, npad), 0)
        jj = lax.broadcasted_iota(I32, (n, npad), 1)
        gt = (srow > scol)
        eq = (srow == scol)
        rank_c = jnp.sum(gt.astype(F32) + (eq & (jj < ii)).astype(F32),
                         axis=1, keepdims=True)
        sel_c = rank_c < k
        mapv = jnp.where(sel_c, g * k + rank_c.astype(I32), -1)
        map_r[...] = mapv

        tt = lax.broadcasted_iota(I32, (n, kpad), 1)
        onehot = (mapv == g * k + tt)
        jglob = (lax.broadcasted_iota(I32, (n, kpad), 0)
                 + g * n).astype(F32)
        inv_r[...] = jnp.sum(jnp.where(onehot, jglob, 0.0),
                             axis=0, keepdims=True).astype(I32)[None]

        scaled = hv * scol
        hs = jnp.where(sel_c, scaled, 0.0)
        hs_r[...] = hs
        maxv = jnp.max(jnp.where(sel_c, scaled, -1e30), axis=0,
                       keepdims=True)
        meanv = jnp.sum(hs, axis=0, keepdims=True) * (1.0 / k)
        ro_r[...] = jnp.concatenate([maxv, meanv], axis=1)[None]

    mapc, inv3, ro3, hs = pl.pallas_call(
        body,
        grid=(G,),
        in_specs=[
            pl.BlockSpec((1, 1, npad), lambda g: (g, 0, 0)),
            pl.BlockSpec((n, 1), lambda g: (g, 0)),
            pl.BlockSpec((n, 128), lambda g: (g, 0)),
        ],
        out_specs=[
            pl.BlockSpec((n, 1), lambda g: (g, 0)),
            pl.BlockSpec((1, 1, kpad), lambda g: (g, 0, 0)),
            pl.BlockSpec((1, 1, 256), lambda g: (g, 0, 0)),
            pl.BlockSpec((n, 128), lambda g: (g, 0)),
        ],
        out_shape=[
            jax.ShapeDtypeStruct((G * n, 1), I32),
            jax.ShapeDtypeStruct((G, 1, kpad), I32),
            jax.ShapeDtypeStruct((G, 1, 256), F32),
            jax.ShapeDtypeStruct((G * n, 128), F32),
        ],
    )(s_row[:, None, :], s_col, h)
    return mapc, inv3[:, 0], ro3[:, 0], hs


def _tc_head(r1, r2, r3, w1t, b1, w2t, b2, w3t, b3):
    def body(r1_r, r2_r, r3_r, w1_r, b1_r, w2_r, b2_r, w3_r, b3_r, o_r):
        z = r1_r[...] + r2_r[...] + r3_r[...]
        z = jnp.maximum(jnp.dot(z, w1_r[...], preferred_element_type=F32)
                        + b1_r[...], 0.0)
        z = jnp.maximum(jnp.dot(z, w2_r[...], preferred_element_type=F32)
                        + b2_r[...], 0.0)
        z3 = jnp.dot(z, w3_r[...], preferred_element_type=F32) + b3_r[...]
        col = lax.broadcasted_iota(I32, (64, 128), 1)
        logits = jnp.where(col < 7, z3, -1e30)
        m = jnp.max(logits, axis=1, keepdims=True)
        lse = jnp.log(jnp.sum(jnp.exp(logits - m), axis=1,
                              keepdims=True)) + m
        o_r[...] = logits - lse

    def full(shape):
        return pl.BlockSpec(shape, lambda: (0, 0))

    return pl.pallas_call(
        body,
        in_specs=[full((64, 256)), full((64, 256)), full((64, 256)),
                  full((256, 128)), full((1, 128)),
                  full((128, 128)), full((1, 128)),
                  full((128, 128)), full((1, 128))],
        out_specs=full((64, 128)),
        out_shape=jax.ShapeDtypeStruct((64, 128), F32),
    )(r1, r2, r3, w1t, b1, w2t, b2, w3t, b3)


def kernel(x, edge_index, batch, W_rel1, b_rel1, W_root1, w_pool1,
           W_rel2, b_rel2, W_root2, w_pool2, W_rel3, b_rel3, W_root3,
           lin1_w, lin1_b, lin2_w, lin2_b, lin3_w, lin3_b):
    pad_ids = D1 + (jnp.arange(EPAD - E, dtype=I32) % 128)
    src = jnp.concatenate([edge_index[0], pad_ids])
    dst = jnp.concatenate([edge_index[1], pad_ids])

    x1t = jnp.zeros((R1, 16), F32).at[:N, :4].set(x)
    wr1 = jnp.zeros((16, 128), F32).at[:4].set(W_rel1.T)
    wx1 = jnp.zeros((16, 128), F32).at[:4].set(W_root1.T)
    cfull = jnp.full((NWORK, 16), EPAD // NWORK, I32)
    parts1 = _sc_segsum(x1t, src, dst, cfull)
    h1, s1 = _tc_conv(parts1[0], parts1[1], x1t, wr1, wx1,
                      b_rel1[None, :], w_pool1[:, None])
    s1row = jnp.full((G, 256), -2.0, F32).at[:, :N0].set(
        s1[:N, 0].reshape(G, N0))
    map1, inv1, ro1, hs1 = _tc_pool(s1row, s1[:N], h1[:N],
                                    n=N0, npad=256, k=K1)
    mapping1 = jnp.full((R1,), -1, I32).at[:N].set(map1[:, 0])
    invf1 = jnp.concatenate([inv1[:, :K1].reshape(-1),
                             jnp.full((R2 - G * K1,), N, I32)])
    hs1p = jnp.zeros((R1, 128), F32).at[:N].set(hs1)

    x2t, ns2, nd2, cnt2 = _sc_pool_apply(hs1p, invf1, mapping1, src, dst,
                                         cfull, rn=R2, dummy_next=D2)
    parts2 = _sc_segsum(x2t, ns2, nd2, cnt2)
    h2, s2 = _tc_conv(parts2[0], parts2[1], x2t, W_rel2.T, W_root2.T,
                      b_rel2[None, :], w_pool2[:, None])
    s2row = jnp.full((G, 256), -2.0, F32).at[:, :K1].set(
        s2[:G * K1, 0].reshape(G, K1))
    map2, inv2, ro2, hs2 = _tc_pool(s2row, s2[:G * K1], h2[:G * K1],
                                    n=K1, npad=256, k=K2)
    mapping2 = jnp.full((R2,), -1, I32).at[:G * K1].set(map2[:, 0])
    invf2 = jnp.concatenate([inv2[:, :K2].reshape(-1),
                             jnp.full((R3 - G * K2,), D2, I32)])
    hs2p = jnp.zeros((R2, 128), F32).at[:G * K1].set(hs2)

    x3t, ns3, nd3, cnt3 = _sc_pool_apply(hs2p, invf2, mapping2, ns2, nd2,
                                         cnt2, rn=R3, dummy_next=D3)
    parts3 = _sc_segsum(x3t, ns3, nd3, cnt3)
    h3, s3 = _tc_conv(parts3[0], parts3[1], x3t, W_rel3.T, W_root3.T,
                      b_rel3[None, :], w_pool2[:, None])
    s3row = s3[:G * K2, 0].reshape(G, K2)
    _, _, ro3, _ = _tc_pool(s3row, s3[:G * K2], h3[:G * K2],
                            n=K2, npad=K2, k=K3)

    def pad64(r):
        return jnp.zeros((64, 256), F32).at[:G].set(r)

    w2t = jnp.zeros((128, 128), F32).at[:, :64].set(lin2_w.T)
    b2p = jnp.zeros((1, 128), F32).at[0, :64].set(lin2_b)
    w3t = jnp.zeros((128, 128), F32).at[:64, :7].set(lin3_w.T)
    b3p = jnp.zeros((1, 128), F32).at[0, :7].set(lin3_b)
    out = _tc_head(pad64(ro1), pad64(ro2), pad64(ro3),
                   lin1_w.T, lin1_b[None, :], w2t, b2p, w3t, b3p)
    return out[:G, :7]
